# baseline (speedup 1.0000x reference)
import functools
import jax
import jax.numpy as jnp
from jax import lax
from jax.experimental import pallas as pl
from jax.experimental.pallas import tpu as pltpu
from jax.experimental.pallas import tpu_sc as plsc


_ROWS = 16384
_COLS = 4096
_K = 128
_TC_ROWS = 12288
_SC_ROWS = _ROWS - _TC_ROWS

_BLOCK_R = 512

_NC = 2
_NS = 16
_NW = _NC * _NS
_RPW = _SC_ROWS // _NW
_G = 8


def _select_body(x_ref, s_ref, o_ref):
    o_ref[...] = jnp.dot(
        x_ref[...], s_ref[...], preferred_element_type=jnp.float32
    )


def _sc_body(x_hbm, w_hbm, out_hbm, row_v, out_stage, w_v):
    wid = lax.axis_index("s") * _NC + lax.axis_index("c")
    row0 = _TC_ROWS + wid * _RPW

    pltpu.sync_copy(w_hbm, w_v)

    def group(gi, _):
        r = row0 + gi * _G
        pltpu.sync_copy(x_hbm.at[pl.ds(r, _G)], row_v)
        for s in range(_G):
            sidx = jnp.full((16,), s, jnp.int32)
            for j in range(_K // 16):
                cidx = w_v[pl.ds(16 * j, 16)]
                vals = plsc.load_gather(row_v, [sidx, cidx])
                out_stage[s, pl.ds(16 * j, 16)] = vals
        pltpu.sync_copy(
            out_stage, out_hbm.at[pl.ds(r - _TC_ROWS, _G)]
        )
        return ()

    lax.fori_loop(0, _RPW // _G, group, ())


def kernel(x, w):
    b, srows, cols = x.shape
    k = w.shape[0]
    xf = x.reshape(b * srows, cols)

    mesh = plsc.VectorSubcoreMesh(core_axis_name="c", subcore_axis_name="s")
    sc_call = functools.partial(
        pl.kernel,
        mesh=mesh,
        compiler_params=pltpu.CompilerParams(needs_layout_passes=False),
        out_type=jax.ShapeDtypeStruct((_SC_ROWS, k), jnp.float32),
        scratch_types=[
            pltpu.VMEM((_G, _COLS), jnp.float32),
            pltpu.VMEM((_G, _K), jnp.float32),
            pltpu.VMEM((_K,), jnp.int32),
        ],
    )(_sc_body)
    out_sc = sc_call(xf, w)

    sel = (
        jax.lax.broadcasted_iota(jnp.int32, (cols, k), 0) == w[None, :]
    ).astype(jnp.float32)
    out_tc = pl.pallas_call(
        _select_body,
        grid=(_TC_ROWS // _BLOCK_R,),
        in_specs=[
            pl.BlockSpec((_BLOCK_R, cols), lambda i: (i, 0)),
            pl.BlockSpec((cols, k), lambda i: (0, 0)),
        ],
        out_specs=pl.BlockSpec((_BLOCK_R, k), lambda i: (i, 0)),
        out_shape=jax.ShapeDtypeStruct((_TC_ROWS, k), jnp.float32),
    )(xf, sel)

    out = jnp.concatenate([out_tc, out_sc], axis=0)
    return out.reshape(b, srows, k)


# speedup vs baseline: 1.0116x; 1.0116x over previous
import functools
import jax
import jax.numpy as jnp
from jax import lax
from jax.experimental import pallas as pl
from jax.experimental.pallas import tpu as pltpu
from jax.experimental.pallas import tpu_sc as plsc


_ROWS = 16384
_COLS = 4096
_K = 128
_TC_ROWS = 15360
_SC_ROWS = _ROWS - _TC_ROWS

_BLOCK_R = 512

_NC = 2
_NS = 16
_NW = _NC * _NS
_RPW = _SC_ROWS // _NW
_G = 8


def _select_body(x_ref, s_ref, o_ref):
    o_ref[...] = jnp.dot(
        x_ref[...], s_ref[...], preferred_element_type=jnp.float32
    )


def _sc_body(x_hbm, w_hbm, out_hbm, row_v, out_stage, w_v):
    wid = lax.axis_index("s") * _NC + lax.axis_index("c")
    row0 = _TC_ROWS + wid * _RPW

    pltpu.sync_copy(w_hbm, w_v)

    def group(gi, _):
        r = row0 + gi * _G
        pltpu.sync_copy(x_hbm.at[pl.ds(r, _G)], row_v)
        for s in range(_G):
            sidx = jnp.full((16,), s, jnp.int32)
            for j in range(_K // 16):
                cidx = w_v[pl.ds(16 * j, 16)]
                vals = plsc.load_gather(row_v, [sidx, cidx])
                out_stage[s, pl.ds(16 * j, 16)] = vals
        pltpu.sync_copy(
            out_stage, out_hbm.at[pl.ds(r - _TC_ROWS, _G)]
        )
        return ()

    lax.fori_loop(0, _RPW // _G, group, ())


def kernel(x, w):
    b, srows, cols = x.shape
    k = w.shape[0]
    xf = x.reshape(b * srows, cols)

    mesh = plsc.VectorSubcoreMesh(core_axis_name="c", subcore_axis_name="s")
    sc_call = functools.partial(
        pl.kernel,
        mesh=mesh,
        compiler_params=pltpu.CompilerParams(needs_layout_passes=False),
        out_type=jax.ShapeDtypeStruct((_SC_ROWS, k), jnp.float32),
        scratch_types=[
            pltpu.VMEM((_G, _COLS), jnp.float32),
            pltpu.VMEM((_G, _K), jnp.float32),
            pltpu.VMEM((_K,), jnp.int32),
        ],
    )(_sc_body)
    out_sc = sc_call(xf, w)

    sel = (
        jax.lax.broadcasted_iota(jnp.int32, (cols, k), 0) == w[None, :]
    ).astype(jnp.float32)
    out_tc = pl.pallas_call(
        _select_body,
        grid=(_TC_ROWS // _BLOCK_R,),
        in_specs=[
            pl.BlockSpec((_BLOCK_R, cols), lambda i: (i, 0)),
            pl.BlockSpec((cols, k), lambda i: (0, 0)),
        ],
        out_specs=pl.BlockSpec((_BLOCK_R, k), lambda i: (i, 0)),
        out_shape=jax.ShapeDtypeStruct((_TC_ROWS, k), jnp.float32),
    )(xf, sel)

    out = jnp.concatenate([out_tc, out_sc], axis=0)
    return out.reshape(b, srows, k)


# speedup vs baseline: 1.2684x; 1.2538x over previous
import jax
import jax.numpy as jnp
from jax.experimental import pallas as pl
from jax.experimental.pallas import tpu as pltpu


_BLOCK_R = 512


def _select_body(x_ref, s_ref, o_ref):
    o_ref[...] = jnp.dot(
        x_ref[...], s_ref[...], preferred_element_type=jnp.float32
    )


def kernel(x, w):
    b, srows, cols = x.shape
    k = w.shape[0]
    xf = x.reshape(b * srows, cols)
    sel = (
        jax.lax.broadcasted_iota(jnp.int32, (cols, k), 0) == w[None, :]
    ).astype(jnp.float32)

    grid = (xf.shape[0] // _BLOCK_R,)
    out = pl.pallas_call(
        _select_body,
        grid=grid,
        in_specs=[
            pl.BlockSpec((_BLOCK_R, cols), lambda i: (i, 0)),
            pl.BlockSpec((cols, k), lambda i: (0, 0)),
        ],
        out_specs=pl.BlockSpec((_BLOCK_R, k), lambda i: (i, 0)),
        out_shape=jax.ShapeDtypeStruct((xf.shape[0], k), jnp.float32),
        compiler_params=pltpu.CompilerParams(
            vmem_limit_bytes=100 * 1024 * 1024,
        ),
    )(xf, sel)
    return out.reshape(b, srows, k)
